# baseline (speedup 1.0000x reference)
import jax
import jax.numpy as jnp
from jax import lax
from jax.experimental import pallas as pl
from jax.experimental.pallas import tpu as pltpu
from jax.experimental.pallas import tpu_sc as plsc

N_NODES = 10000
E = 320000
D = 128
NC = 2
NS = 16
NW = NC * NS
CHUNK = 80
TOTAL_CH = E // CHUNK
NCHW = 125
MAIN = 124
NBUF = 4
IBUF = 8
DBUF = 3
CHUNKD = 128
NCHD = 78
TAILD = E // CHUNKD - NW * NCHD
N_PAD = 10112
RPT = N_PAD // NS
N_PAD_DEG = 10240
RPT_DEG = N_PAD_DEG // NS


def _sc_mesh():
    return plsc.VectorSubcoreMesh(
        core_axis_name="c", subcore_axis_name="s", num_cores=NC, num_subcores=NS
    )


def _deg_body(edges_hbm, zrow_hbm, deg_out, idx_v, ones_v, sem_i, sem_s,
              deg_sp):
    c = lax.axis_index("c")
    s = lax.axis_index("s")
    w = c * NS + s
    base = w * NCHD
    pltpu.sync_copy(zrow_hbm, deg_sp.at[pl.ds(s * RPT_DEG, RPT_DEG)])
    for i in range(CHUNKD // 16):
        ones_v[pl.ds(i * 16, 16)] = jnp.ones((16,), jnp.float32)

    def start_idx(j, b):
        pltpu.async_copy(
            edges_hbm.at[:, pl.ds((base + j) * CHUNKD, CHUNKD)],
            idx_v.at[b], sem_i.at[b])

    def wait_idx(j, b):
        pltpu.make_async_copy(
            edges_hbm.at[:, pl.ds((base + j) * CHUNKD, CHUNKD)],
            idx_v.at[b], sem_i.at[b]).wait()

    def start_scat(b):
        pltpu.async_copy(ones_v, deg_sp.at[idx_v.at[b, 1]], sem_s.at[b],
                         add=True)

    def wait_scat(b):
        pltpu.make_async_copy(ones_v, deg_sp.at[idx_v.at[b, 1]],
                              sem_s.at[b]).wait()

    start_idx(0, 0)
    start_idx(1, 1)
    plsc.subcore_barrier()

    @pl.loop(0, NCHD, step=DBUF)
    def _(g):
        for b in range(DBUF):
            j = g + b
            wait_idx(j, b)
            start_scat(b)

            @pl.when(j >= 1)
            def _():
                wait_scat((b - 1) % DBUF)

            @pl.when(j + 2 < NCHD)
            def _():
                start_idx(j + 2, (b + 2) % DBUF)

    wait_scat((NCHD - 1) % DBUF)

    @pl.when(w < TAILD)
    def _():
        pltpu.sync_copy(
            edges_hbm.at[:, pl.ds((NW * NCHD + w) * CHUNKD, CHUNKD)],
            idx_v.at[0])
        pltpu.sync_copy(ones_v, deg_sp.at[idx_v.at[0, 1]], add=True)

    plsc.subcore_barrier()
    pltpu.sync_copy(deg_sp.at[pl.ds(s * RPT_DEG, RPT_DEG)], deg_out.at[c, s])


def _degree_pass(edge_index):
    zrow = jnp.zeros((RPT_DEG,), jnp.float32)
    k = pl.kernel(
        _deg_body,
        out_type=jax.ShapeDtypeStruct((NC, NS, RPT_DEG), jnp.float32),
        mesh=_sc_mesh(),
        scratch_types=[
            pltpu.VMEM((DBUF, 2, CHUNKD), jnp.int32),
            pltpu.VMEM((CHUNKD,), jnp.float32),
            pltpu.SemaphoreType.DMA((DBUF,)),
            pltpu.SemaphoreType.DMA((DBUF,)),
            pltpu.VMEM_SHARED((N_PAD_DEG,), jnp.float32),
        ],
    )
    return k(edge_index, zrow)


def _scat_body(y_hbm, srcl_hbm, dstl_hbm, ztile_hbm, acc_out,
               src_v, dst_v, rows_v, sem_i, sem_g, sem_s, acc_sp):
    c = lax.axis_index("c")
    s = lax.axis_index("s")
    w = c * NS + s
    base = w * NCHW
    pltpu.sync_copy(ztile_hbm, acc_sp.at[pl.ds(s * RPT, RPT)])

    def start_idx(j):
        pltpu.async_copy(srcl_hbm.at[pl.ds((base + j) * CHUNK, CHUNK)],
                         src_v.at[j % IBUF], sem_i.at[j % NBUF])
        pltpu.async_copy(dstl_hbm.at[pl.ds((base + j) * CHUNK, CHUNK)],
                         dst_v.at[j % IBUF], sem_i.at[j % NBUF])

    def wait_idx(j):
        pltpu.make_async_copy(
            srcl_hbm.at[pl.ds((base + j) * CHUNK, CHUNK)],
            src_v.at[j % IBUF], sem_i.at[j % NBUF]).wait()
        pltpu.make_async_copy(
            dstl_hbm.at[pl.ds((base + j) * CHUNK, CHUNK)],
            dst_v.at[j % IBUF], sem_i.at[j % NBUF]).wait()

    def start_gather(j, r):
        pltpu.async_copy(y_hbm.at[src_v.at[j % IBUF]], rows_v.at[r],
                         sem_g.at[r])

    def wait_gather(j, r):
        pltpu.make_async_copy(y_hbm.at[src_v.at[j % IBUF]], rows_v.at[r],
                              sem_g.at[r]).wait()

    def start_scatter(j, r):
        pltpu.async_copy(rows_v.at[r], acc_sp.at[dst_v.at[j % IBUF]],
                         sem_s.at[r], add=True)

    def wait_scatter(j, r):
        pltpu.make_async_copy(rows_v.at[r], acc_sp.at[dst_v.at[j % IBUF]],
                              sem_s.at[r]).wait()

    for b in range(4):
        start_idx(b)
    for b in range(3):
        wait_idx(b)
        start_gather(b, b)
    plsc.subcore_barrier()

    @pl.loop(0, MAIN, step=NBUF)
    def _(g):
        for b in range(NBUF):
            j = g + b
            r = b
            wait_gather(j, r)
            start_scatter(j, r)

            @pl.when(j >= 1)
            def _():
                wait_scatter(j - 1, (r - 1) % NBUF)

            @pl.when(j + 3 < MAIN)
            def _():
                wait_idx(j + 3)
                start_gather(j + 3, (r + 3) % NBUF)

            @pl.when(j + 4 < MAIN)
            def _():
                start_idx(j + 4)

    wait_scatter(MAIN - 1, (MAIN - 1) % NBUF)

    ch = base + MAIN
    pltpu.sync_copy(srcl_hbm.at[pl.ds(ch * CHUNK, CHUNK)], src_v.at[0])
    pltpu.sync_copy(dstl_hbm.at[pl.ds(ch * CHUNK, CHUNK)], dst_v.at[0])
    pltpu.async_copy(y_hbm.at[src_v.at[0]], rows_v.at[0],
                     sem_g.at[0]).wait()
    pltpu.sync_copy(rows_v.at[0], acc_sp.at[dst_v.at[0]], add=True)

    plsc.subcore_barrier()
    pltpu.sync_copy(acc_sp.at[pl.ds(s * RPT, RPT)], acc_out.at[c, s])


def _scatter_pass(y, srcl, dstl):
    ztile = jnp.zeros((RPT, D), jnp.float32)
    k = pl.kernel(
        _scat_body,
        out_type=jax.ShapeDtypeStruct((NC, NS, RPT, D), jnp.float32),
        mesh=_sc_mesh(),
        scratch_types=[
            pltpu.VMEM((IBUF, CHUNK), jnp.int32),
            pltpu.VMEM((IBUF, CHUNK), jnp.int32),
            pltpu.VMEM((NBUF, CHUNK, D), jnp.float32),
            pltpu.SemaphoreType.DMA((IBUF,)),
            pltpu.SemaphoreType.DMA((NBUF,)),
            pltpu.SemaphoreType.DMA((NBUF,)),
            pltpu.VMEM_SHARED((N_PAD, D), jnp.float32),
        ],
    )
    return k(y, srcl, dstl, ztile)


BLK = 2000
DBLK = 1280


def _mm_body(x_ref, w_ref, xw_ref):
    xw_ref[...] = jnp.dot(x_ref[...], w_ref[...],
                          preferred_element_type=jnp.float32)


def _tc_mm(x, W):
    return pl.pallas_call(
        _mm_body,
        grid=(N_NODES // BLK,),
        in_specs=[
            pl.BlockSpec((BLK, D), lambda i: (i, 0)),
            pl.BlockSpec((D, D), lambda i: (0, 0)),
        ],
        out_specs=pl.BlockSpec((BLK, D), lambda i: (i, 0)),
        out_shape=jax.ShapeDtypeStruct((N_NODES, D), jnp.float32),
    )(x, W)


def _dinv_body(deg_ref, dinv_ref):
    d = deg_ref[0:1, :] + deg_ref[1:2, :] + 1.0
    dinv_ref[...] = lax.rsqrt(d).reshape(DBLK, 1)


def _tc_dinv(deg_parts):
    return pl.pallas_call(
        _dinv_body,
        grid=(N_PAD_DEG // DBLK,),
        in_specs=[pl.BlockSpec((NC, DBLK), lambda i: (0, i))],
        out_specs=pl.BlockSpec((DBLK, 1), lambda i: (i, 0)),
        out_shape=jax.ShapeDtypeStruct((N_PAD_DEG, 1), jnp.float32),
    )(deg_parts)


def _scale_body(xw_ref, dinv_ref, y_ref):
    y_ref[...] = dinv_ref[...] * xw_ref[...]


def _tc_scale(xw, dinv):
    return pl.pallas_call(
        _scale_body,
        grid=(N_NODES // BLK,),
        in_specs=[
            pl.BlockSpec((BLK, D), lambda i: (i, 0)),
            pl.BlockSpec((BLK, 1), lambda i: (i, 0)),
        ],
        out_specs=pl.BlockSpec((BLK, D), lambda i: (i, 0)),
        out_shape=jax.ShapeDtypeStruct((N_NODES, D), jnp.float32),
    )(xw, dinv)


def _mid_body(acc_ref, y1_ref, dinv_ref, b_ref, w_ref, y2_ref):
    dinv = dinv_ref[...]
    h = dinv * (acc_ref[0] + acc_ref[1] + y1_ref[...]) + b_ref[...]
    h = jnp.maximum(h, 0.0)
    y2_ref[...] = dinv * jnp.dot(h, w_ref[...],
                                 preferred_element_type=jnp.float32)


def _tc_mid(acc, y1, dinv, b1, W2):
    return pl.pallas_call(
        _mid_body,
        grid=(N_NODES // BLK,),
        in_specs=[
            pl.BlockSpec((NC, BLK, D), lambda i: (0, i, 0)),
            pl.BlockSpec((BLK, D), lambda i: (i, 0)),
            pl.BlockSpec((BLK, 1), lambda i: (i, 0)),
            pl.BlockSpec((1, D), lambda i: (0, 0)),
            pl.BlockSpec((D, D), lambda i: (0, 0)),
        ],
        out_specs=pl.BlockSpec((BLK, D), lambda i: (i, 0)),
        out_shape=jax.ShapeDtypeStruct((N_NODES, D), jnp.float32),
    )(acc, y1, dinv, b1, W2)


def _fin_body(acc_ref, y2_ref, dinv_ref, b_ref, z_ref):
    z_ref[...] = (dinv_ref[...] * (acc_ref[0] + acc_ref[1] + y2_ref[...])
                  + b_ref[...])


def _tc_final(acc, y2, dinv, b2):
    return pl.pallas_call(
        _fin_body,
        grid=(N_NODES // BLK,),
        in_specs=[
            pl.BlockSpec((NC, BLK, D), lambda i: (0, i, 0)),
            pl.BlockSpec((BLK, D), lambda i: (i, 0)),
            pl.BlockSpec((BLK, 1), lambda i: (i, 0)),
            pl.BlockSpec((1, D), lambda i: (0, 0)),
        ],
        out_specs=pl.BlockSpec((BLK, D), lambda i: (i, 0)),
        out_shape=jax.ShapeDtypeStruct((N_NODES, D), jnp.float32),
    )(acc, y2, dinv, b2)


def kernel(x, edge_index, W1, b1, W2, b2):
    b1r = b1.reshape(1, D)
    b2r = b2.reshape(1, D)

    srcl = edge_index[0]
    dstl = edge_index[1]
    deg_parts = _degree_pass(edge_index).reshape(NC, N_PAD_DEG)
    xw1 = _tc_mm(x, W1)
    dinv = _tc_dinv(deg_parts)[:N_NODES]
    y1 = _tc_scale(xw1, dinv)
    acc1 = _scatter_pass(y1, srcl, dstl).reshape(NC, N_PAD, D)
    y2 = _tc_mid(acc1, y1, dinv, b1r, W2)
    acc2 = _scatter_pass(y2, srcl, dstl).reshape(NC, N_PAD, D)
    return _tc_final(acc2, y2, dinv, b2r)


# speedup vs baseline: 1.0163x; 1.0163x over previous
import jax
import jax.numpy as jnp
from jax import lax
from jax.experimental import pallas as pl
from jax.experimental.pallas import tpu as pltpu
from jax.experimental.pallas import tpu_sc as plsc

N_NODES = 10000
E = 320000
D = 128
NC = 2
NS = 16
NW = NC * NS
CHUNK = 80
TOTAL_CH = E // CHUNK
NCHW = 125
MAIN = 124
NBUF = 4
IBUF = 8
DBUF = 3
CHUNKD = 128
NCHD = 78
TAILD = E // CHUNKD - NW * NCHD
N_PAD = 10112
RPT = N_PAD // NS
N_PAD_DEG = 10240
RPT_DEG = N_PAD_DEG // NS


def _sc_mesh():
    return plsc.VectorSubcoreMesh(
        core_axis_name="c", subcore_axis_name="s", num_cores=NC, num_subcores=NS
    )


def _deg_body(edges_hbm, zrow_hbm, deg_out, idx_v, ones_v, sem_i, sem_s,
              deg_sp):
    c = lax.axis_index("c")
    s = lax.axis_index("s")
    w = c * NS + s
    base = w * NCHD
    pltpu.sync_copy(zrow_hbm, deg_sp.at[pl.ds(s * RPT_DEG, RPT_DEG)])
    for i in range(CHUNKD // 16):
        ones_v[pl.ds(i * 16, 16)] = jnp.ones((16,), jnp.float32)

    def start_idx(j, b):
        pltpu.async_copy(
            edges_hbm.at[:, pl.ds((base + j) * CHUNKD, CHUNKD)],
            idx_v.at[b], sem_i.at[b])

    def wait_idx(j, b):
        pltpu.make_async_copy(
            edges_hbm.at[:, pl.ds((base + j) * CHUNKD, CHUNKD)],
            idx_v.at[b], sem_i.at[b]).wait()

    def start_scat(b):
        pltpu.async_copy(ones_v, deg_sp.at[idx_v.at[b, 1]], sem_s.at[b],
                         add=True)

    def wait_scat(b):
        pltpu.make_async_copy(ones_v, deg_sp.at[idx_v.at[b, 1]],
                              sem_s.at[b]).wait()

    start_idx(0, 0)
    start_idx(1, 1)
    plsc.subcore_barrier()

    @pl.loop(0, NCHD, step=DBUF)
    def _(g):
        for b in range(DBUF):
            j = g + b
            wait_idx(j, b)
            start_scat(b)

            @pl.when(j >= 1)
            def _():
                wait_scat((b - 1) % DBUF)

            @pl.when(j + 2 < NCHD)
            def _():
                start_idx(j + 2, (b + 2) % DBUF)

    wait_scat((NCHD - 1) % DBUF)

    @pl.when(w < TAILD)
    def _():
        pltpu.sync_copy(
            edges_hbm.at[:, pl.ds((NW * NCHD + w) * CHUNKD, CHUNKD)],
            idx_v.at[0])
        pltpu.sync_copy(ones_v, deg_sp.at[idx_v.at[0, 1]], add=True)

    plsc.subcore_barrier()
    pltpu.sync_copy(deg_sp.at[pl.ds(s * RPT_DEG, RPT_DEG)], deg_out.at[c, s])


def _degree_pass(edge_index):
    zrow = jnp.zeros((RPT_DEG,), jnp.float32)
    k = pl.kernel(
        _deg_body,
        out_type=jax.ShapeDtypeStruct((NC, NS, RPT_DEG), jnp.float32),
        mesh=_sc_mesh(),
        scratch_types=[
            pltpu.VMEM((DBUF, 2, CHUNKD), jnp.int32),
            pltpu.VMEM((CHUNKD,), jnp.float32),
            pltpu.SemaphoreType.DMA((DBUF,)),
            pltpu.SemaphoreType.DMA((DBUF,)),
            pltpu.VMEM_SHARED((N_PAD_DEG,), jnp.float32),
        ],
    )
    return k(edge_index, zrow)


def _scat_body(y_hbm, srcl_hbm, dstl_hbm, ztile_hbm, acc_out,
               src_v, dst_v, rows_v, sem_i, sem_g, sem_s, acc_sp):
    c = lax.axis_index("c")
    s = lax.axis_index("s")
    w = c * NS + s
    base = w * NCHW
    pltpu.sync_copy(ztile_hbm, acc_sp.at[pl.ds(s * RPT, RPT)])

    def start_idx(j):
        pltpu.async_copy(srcl_hbm.at[pl.ds((base + j) * CHUNK, CHUNK)],
                         src_v.at[j % IBUF], sem_i.at[j % NBUF])
        pltpu.async_copy(dstl_hbm.at[pl.ds((base + j) * CHUNK, CHUNK)],
                         dst_v.at[j % IBUF], sem_i.at[j % NBUF])

    def wait_idx(j):
        pltpu.make_async_copy(
            srcl_hbm.at[pl.ds((base + j) * CHUNK, CHUNK)],
            src_v.at[j % IBUF], sem_i.at[j % NBUF]).wait()
        pltpu.make_async_copy(
            dstl_hbm.at[pl.ds((base + j) * CHUNK, CHUNK)],
            dst_v.at[j % IBUF], sem_i.at[j % NBUF]).wait()

    def start_gather(j, r):
        pltpu.async_copy(y_hbm.at[src_v.at[j % IBUF]], rows_v.at[r],
                         sem_g.at[r])

    def wait_gather(j, r):
        pltpu.make_async_copy(y_hbm.at[src_v.at[j % IBUF]], rows_v.at[r],
                              sem_g.at[r]).wait()

    def start_scatter(j, r):
        pltpu.async_copy(rows_v.at[r], acc_sp.at[dst_v.at[j % IBUF]],
                         sem_s.at[r], add=True)

    def wait_scatter(j, r):
        pltpu.make_async_copy(rows_v.at[r], acc_sp.at[dst_v.at[j % IBUF]],
                              sem_s.at[r]).wait()

    for b in range(4):
        start_idx(b)
    for b in range(3):
        wait_idx(b)
        start_gather(b, b)
    plsc.subcore_barrier()

    @pl.loop(0, MAIN, step=NBUF)
    def _(g):
        for b in range(NBUF):
            j = g + b
            r = b
            wait_gather(j, r)
            start_scatter(j, r)

            @pl.when(j >= 1)
            def _():
                wait_scatter(j - 1, (r - 1) % NBUF)

            @pl.when(j + 3 < MAIN)
            def _():
                wait_idx(j + 3)
                start_gather(j + 3, (r + 3) % NBUF)

            @pl.when(j + 4 < MAIN)
            def _():
                start_idx(j + 4)

    wait_scatter(MAIN - 1, (MAIN - 1) % NBUF)

    ch = base + MAIN
    pltpu.sync_copy(srcl_hbm.at[pl.ds(ch * CHUNK, CHUNK)], src_v.at[0])
    pltpu.sync_copy(dstl_hbm.at[pl.ds(ch * CHUNK, CHUNK)], dst_v.at[0])
    pltpu.async_copy(y_hbm.at[src_v.at[0]], rows_v.at[0],
                     sem_g.at[0]).wait()
    pltpu.sync_copy(rows_v.at[0], acc_sp.at[dst_v.at[0]], add=True)

    plsc.subcore_barrier()
    pltpu.sync_copy(acc_sp.at[pl.ds(s * RPT, RPT)], acc_out.at[c, s])


def _scatter_pass(y, srcl, dstl):
    ztile = jnp.zeros((RPT, D), jnp.float32)
    k = pl.kernel(
        _scat_body,
        out_type=jax.ShapeDtypeStruct((NC, NS, RPT, D), jnp.float32),
        mesh=_sc_mesh(),
        scratch_types=[
            pltpu.VMEM((IBUF, CHUNK), jnp.int32),
            pltpu.VMEM((IBUF, CHUNK), jnp.int32),
            pltpu.VMEM((NBUF, CHUNK, D), jnp.float32),
            pltpu.SemaphoreType.DMA((IBUF,)),
            pltpu.SemaphoreType.DMA((NBUF,)),
            pltpu.SemaphoreType.DMA((NBUF,)),
            pltpu.VMEM_SHARED((N_PAD, D), jnp.float32),
        ],
    )
    return k(y, srcl, dstl, ztile)


BLK = 2000
DBLK = 1280


def _mm_body(x_ref, w_ref, xw_ref):
    xw_ref[...] = jnp.dot(x_ref[...], w_ref[...],
                          preferred_element_type=jnp.float32)


def _tc_mm(x, W):
    return pl.pallas_call(
        _mm_body,
        grid=(N_NODES // BLK,),
        in_specs=[
            pl.BlockSpec((BLK, D), lambda i: (i, 0)),
            pl.BlockSpec((D, D), lambda i: (0, 0)),
        ],
        out_specs=pl.BlockSpec((BLK, D), lambda i: (i, 0)),
        out_shape=jax.ShapeDtypeStruct((N_NODES, D), jnp.float32),
    )(x, W)


def _dinv_body(deg_ref, dinv_ref):
    d = deg_ref[0:1, :] + deg_ref[1:2, :] + 1.0
    dinv_ref[...] = lax.rsqrt(d).reshape(DBLK, 1)


def _tc_dinv(deg_parts):
    return pl.pallas_call(
        _dinv_body,
        grid=(N_PAD_DEG // DBLK,),
        in_specs=[pl.BlockSpec((NC, DBLK), lambda i: (0, i))],
        out_specs=pl.BlockSpec((DBLK, 1), lambda i: (i, 0)),
        out_shape=jax.ShapeDtypeStruct((N_PAD_DEG, 1), jnp.float32),
    )(deg_parts)


def _scale_body(xw_ref, dinv_ref, y_ref):
    y_ref[...] = dinv_ref[...] * xw_ref[...]


def _tc_scale(xw, dinv):
    return pl.pallas_call(
        _scale_body,
        grid=(N_NODES // BLK,),
        in_specs=[
            pl.BlockSpec((BLK, D), lambda i: (i, 0)),
            pl.BlockSpec((BLK, 1), lambda i: (i, 0)),
        ],
        out_specs=pl.BlockSpec((BLK, D), lambda i: (i, 0)),
        out_shape=jax.ShapeDtypeStruct((N_NODES, D), jnp.float32),
    )(xw, dinv)


def _mid_body(acc_ref, y1_ref, dinv_ref, b_ref, w_ref, y2_ref):
    dinv = dinv_ref[...]
    h = dinv * (acc_ref[0] + acc_ref[1] + y1_ref[...]) + b_ref[...]
    h = jnp.maximum(h, 0.0)
    y2_ref[...] = dinv * jnp.dot(h, w_ref[...],
                                 preferred_element_type=jnp.float32)


def _tc_mid(acc, y1, dinv, b1, W2):
    return pl.pallas_call(
        _mid_body,
        grid=(N_NODES // BLK,),
        in_specs=[
            pl.BlockSpec((NC, BLK, D), lambda i: (0, i, 0)),
            pl.BlockSpec((BLK, D), lambda i: (i, 0)),
            pl.BlockSpec((BLK, 1), lambda i: (i, 0)),
            pl.BlockSpec((1, D), lambda i: (0, 0)),
            pl.BlockSpec((D, D), lambda i: (0, 0)),
        ],
        out_specs=pl.BlockSpec((BLK, D), lambda i: (i, 0)),
        out_shape=jax.ShapeDtypeStruct((N_NODES, D), jnp.float32),
    )(acc, y1, dinv, b1, W2)


def _fin_body(acc_ref, y2_ref, dinv_ref, b_ref, z_ref):
    z_ref[...] = (dinv_ref[...] * (acc_ref[0] + acc_ref[1] + y2_ref[...])
                  + b_ref[...])


def _tc_final(acc, y2, dinv, b2):
    return pl.pallas_call(
        _fin_body,
        grid=(N_NODES // BLK,),
        in_specs=[
            pl.BlockSpec((NC, BLK, D), lambda i: (0, i, 0)),
            pl.BlockSpec((BLK, D), lambda i: (i, 0)),
            pl.BlockSpec((BLK, 1), lambda i: (i, 0)),
            pl.BlockSpec((1, D), lambda i: (0, 0)),
        ],
        out_specs=pl.BlockSpec((BLK, D), lambda i: (i, 0)),
        out_shape=jax.ShapeDtypeStruct((N_NODES, D), jnp.float32),
    )(acc, y2, dinv, b2)


def kernel(x, edge_index, W1, b1, W2, b2):
    b1r = b1.reshape(1, D)
    b2r = b2.reshape(1, D)

    srcl = edge_index[0]
    dstl = edge_index[1]
    deg_parts = _degree_pass(edge_index).reshape(NC, N_PAD_DEG)
    xw1 = _tc_mm(x, W1)
    dinv = _tc_dinv(deg_parts)
    y1 = _tc_scale(xw1, dinv)
    acc1 = _scatter_pass(y1, srcl, dstl).reshape(NC, N_PAD, D)
    y2 = _tc_mid(acc1, y1, dinv, b1r, W2)
    acc2 = _scatter_pass(y2, srcl, dstl).reshape(NC, N_PAD, D)
    return _tc_final(acc2, y2, dinv, b2r)
